# Initial kernel scaffold; baseline (speedup 1.0000x reference)
import jax
import jax.numpy as jnp
from jax.experimental import pallas as pl


def kernel(atom, bond, adj_matrix, W1, W2, b1, b2):
    raise NotImplementedError("write your pallas kernel here")


# speedup vs baseline: 9.9716x; 9.9716x over previous
import functools

import jax
import jax.numpy as jnp
from jax import lax
from jax.experimental import pallas as pl
from jax.experimental.pallas import tpu as pltpu
from jax.experimental.pallas import tpu_sc as plsc

B, N, M = 2, 10000, 16
F = 128
FB = 16

NC, NS = 2, 16
NW = NC * NS
NPAD = 20480
NODES_W = NPAD // NW
CHUNK = 128
NCHUNK = NODES_W // CHUNK
PER_CORE = NPAD // NC

def _neighbor_sum_body(atom_hbm, adj_hbm, slots_hbm, out_hbm,
                       idx_v, slots_v, stage_v, acc_sh, sem):
    cid = lax.axis_index("c")
    sid = lax.axis_index("s")
    gbase = (cid * NS + sid) * NODES_W
    lbase = sid * NODES_W

    pltpu.sync_copy(adj_hbm.at[:, pl.ds(gbase, NODES_W)], idx_v)
    pltpu.sync_copy(slots_hbm.at[sid], slots_v)

    for c in range(NCHUNK):
        pltpu.async_copy(
            atom_hbm.at[idx_v.at[0, pl.ds(c * CHUNK, CHUNK)]], stage_v, sem
        ).wait()
        pltpu.sync_copy(stage_v, acc_sh.at[pl.ds(lbase + c * CHUNK, CHUNK)])

    def step(t, carry):
        c = t // (M - 1)
        m = t % (M - 1) + 1
        pltpu.async_copy(
            atom_hbm.at[idx_v.at[m, pl.ds(c * CHUNK, CHUNK)]], stage_v, sem
        ).wait()
        pltpu.sync_copy(stage_v, acc_sh.at[slots_v.at[c]], add=True)
        return carry

    lax.fori_loop(0, NCHUNK * (M - 1), step, 0)

    pltpu.sync_copy(acc_sh.at[pl.ds(lbase, NODES_W)],
                    out_hbm.at[pl.ds(gbase, NODES_W)])


@functools.lru_cache(maxsize=None)
def _neighbor_sum_kernel():
    mesh = plsc.VectorSubcoreMesh(core_axis_name="c", subcore_axis_name="s")
    return pl.kernel(
        _neighbor_sum_body,
        out_type=jax.ShapeDtypeStruct((NPAD, F), jnp.float32),
        mesh=mesh,
        scratch_types=[
            pltpu.VMEM((M, NODES_W), jnp.int32),
            pltpu.VMEM((NCHUNK, CHUNK), jnp.int32),
            pltpu.VMEM((CHUNK, F), jnp.float32),
            pltpu.VMEM_SHARED((PER_CORE, F), jnp.float32),
            pltpu.SemaphoreType.DMA,
        ],
    )


def _neighbor_sum(atom_flat, adj_t, slots):
    return _neighbor_sum_kernel()(atom_flat, adj_t, slots)


TB = 1000
TR = 2000


def _p_body(bond_ref, sel_ref, p_ref):
    x = bond_ref[0]
    ssq = jnp.dot(x * x, sel_ref[...], preferred_element_type=jnp.float32)
    w = 1.0 / ssq
    w = w / jnp.maximum(jnp.sum(w, axis=1, keepdims=True), 1e-12)
    w8 = w[:, :8] * w[:, 8:]
    w4 = w8[:, :4] * w8[:, 4:]
    w2 = w4[:, :2] * w4[:, 2:]
    p_ref[...] = (w2[:, 0] * w2[:, 1]).reshape(1, 1, TB)


def _r_body(p_ref, r_ref):
    p = p_ref[...]
    q = p / jnp.maximum(jnp.sum(jnp.abs(p), axis=0, keepdims=True), 1e-12)
    s = 1.0 / q
    r_ref[...] = s / jnp.maximum(jnp.sum(jnp.abs(s), axis=1, keepdims=True), 1e-12)


def _out_body(a_ref, nb_ref, r_ref, w_ref, b_ref, o_ref):
    u = (a_ref[...] + nb_ref[...] * (1.0 / M)) * r_ref[...]
    y = jnp.dot(u, w_ref[...], preferred_element_type=jnp.float32) + b_ref[...]
    o_ref[...] = jnp.maximum(y, 0.0)


def kernel(atom, bond, adj_matrix, W1, W2, b1, b2):
    atom_flat = atom.reshape(B * N, F)

    adj = adj_matrix.astype(jnp.int32) + (jnp.arange(B, dtype=jnp.int32) * N)[:, None, None]
    adj_t = adj.reshape(B * N, M).T
    adj_t = jnp.pad(adj_t, ((0, 0), (0, NPAD - B * N)))
    slots = jnp.arange(NS * NODES_W, dtype=jnp.int32).reshape(NS, NCHUNK, CHUNK)

    neigh = _neighbor_sum(atom_flat, adj_t, slots)

    nblk = B * N // TB
    bond_r = bond.reshape(nblk, TB, M * FB)
    sel = jnp.kron(jnp.eye(M, dtype=jnp.float32),
                   jnp.ones((FB, 1), dtype=jnp.float32))
    p = pl.pallas_call(
        _p_body,
        out_shape=jax.ShapeDtypeStruct((nblk, 1, TB), jnp.float32),
        grid=(nblk,),
        in_specs=[
            pl.BlockSpec((1, TB, M * FB), lambda i: (i, 0, 0)),
            pl.BlockSpec((M * FB, M), lambda i: (0, 0)),
        ],
        out_specs=pl.BlockSpec((1, 1, TB), lambda i: (i, 0, 0)),
    )(bond_r, sel).reshape(B, N)

    r = pl.pallas_call(
        _r_body,
        out_shape=jax.ShapeDtypeStruct((B, N), jnp.float32),
    )(p)

    out = pl.pallas_call(
        _out_body,
        out_shape=jax.ShapeDtypeStruct((B * N, F), jnp.float32),
        grid=(B * N // TR,),
        in_specs=[
            pl.BlockSpec((TR, F), lambda i: (i, 0)),
            pl.BlockSpec((TR, F), lambda i: (i, 0)),
            pl.BlockSpec((TR, 1), lambda i: (i, 0)),
            pl.BlockSpec((F, F), lambda i: (0, 0)),
            pl.BlockSpec((1, F), lambda i: (0, 0)),
        ],
        out_specs=pl.BlockSpec((TR, F), lambda i: (i, 0)),
    )(atom_flat, neigh[: B * N], r.reshape(B * N, 1), W1, b1.reshape(1, F))

    return out.reshape(B, N, F)


# speedup vs baseline: 11.1023x; 1.1134x over previous
import functools

import jax
import jax.numpy as jnp
from jax import lax
from jax.experimental import pallas as pl
from jax.experimental.pallas import tpu as pltpu
from jax.experimental.pallas import tpu_sc as plsc

B, N, M = 2, 10000, 16
F = 128
FB = 16

NC, NS = 2, 16
NW = NC * NS
NPAD = 20480
NODES_W = NPAD // NW
CHUNK = 128
NCHUNK = NODES_W // CHUNK
PER_CORE = NPAD // NC

NSLOT = 3
WAVE = NS * CHUNK


def _neighbor_sum_body(atom_hbm, adj_hbm, slots_hbm, out_hbm,
                       idx_v, slots_v, stage_v, acc0, acc1,
                       gsem0, gsem1, gsem2, ssem0, ssem1, ssem2,
                       wsem0, wsem1):
    cid = lax.axis_index("c")
    sid = lax.axis_index("s")
    gbase = (cid * NS + sid) * NODES_W
    gsems = [gsem0, gsem1, gsem2]
    ssems = [ssem0, ssem1, ssem2]
    wsems = [wsem0, wsem1]
    accs = [acc0, acc1]

    pltpu.sync_copy(adj_hbm.at[:, pl.ds(gbase, NODES_W)], idx_v)
    pltpu.sync_copy(slots_hbm.at[sid], slots_v)

    for c in range(NCHUNK):
        acc = accs[c % 2]
        wsem = wsems[c % 2]
        out_slice = out_hbm.at[pl.ds(gbase + c * CHUNK, CHUNK)]
        my_slab = acc.at[pl.ds(sid * CHUNK, CHUNK)]

        def gather(m, slot, sem, c=c):
            return pltpu.async_copy(
                atom_hbm.at[idx_v.at[m, pl.ds(c * CHUNK, CHUNK)]],
                stage_v.at[slot], sem)

        if c >= 2:
            pltpu.make_async_copy(my_slab, out_slice, wsem).wait()

        gather(0, 0, gsems[0]).wait()
        pltpu.sync_copy(stage_v.at[0], my_slab)

        for j in range(NSLOT):
            gather(j + 1, j, gsems[j])

        def step(i, carry, c=c):
            for j in range(NSLOT):
                m = NSLOT * i + j + 1
                pltpu.make_async_copy(
                    atom_hbm.at[idx_v.at[0, pl.ds(c * CHUNK, CHUNK)]],
                    stage_v.at[j], gsems[j]).wait()
                pltpu.async_copy(stage_v.at[j], acc.at[slots_v.at[0]],
                                 ssems[j], add=True).wait()
                gather(m + NSLOT, j, gsems[j])
            return carry

        lax.fori_loop(0, (M - 4) // NSLOT, step, 0)

        for j in range(NSLOT):
            pltpu.make_async_copy(
                atom_hbm.at[idx_v.at[0, pl.ds(c * CHUNK, CHUNK)]],
                stage_v.at[j], gsems[j]).wait()
            pltpu.async_copy(stage_v.at[j], acc.at[slots_v.at[0]],
                             ssems[j], add=True).wait()

        pltpu.async_copy(my_slab, out_slice, wsem)

    for c in (NCHUNK - 2, NCHUNK - 1):
        pltpu.make_async_copy(
            accs[c % 2].at[pl.ds(sid * CHUNK, CHUNK)],
            out_hbm.at[pl.ds(gbase + c * CHUNK, CHUNK)],
            wsems[c % 2]).wait()


@functools.lru_cache(maxsize=None)
def _neighbor_sum_kernel():
    mesh = plsc.VectorSubcoreMesh(core_axis_name="c", subcore_axis_name="s")
    return pl.kernel(
        _neighbor_sum_body,
        out_type=jax.ShapeDtypeStruct((NPAD, F), jnp.float32),
        mesh=mesh,
        scratch_types=[
            pltpu.VMEM((M, NODES_W), jnp.int32),
            pltpu.VMEM((1, CHUNK), jnp.int32),
            pltpu.VMEM((NSLOT, CHUNK, F), jnp.float32),
            pltpu.VMEM_SHARED((WAVE, F), jnp.float32),
            pltpu.VMEM_SHARED((WAVE, F), jnp.float32),
            pltpu.SemaphoreType.DMA,
            pltpu.SemaphoreType.DMA,
            pltpu.SemaphoreType.DMA,
            pltpu.SemaphoreType.DMA,
            pltpu.SemaphoreType.DMA,
            pltpu.SemaphoreType.DMA,
            pltpu.SemaphoreType.DMA,
            pltpu.SemaphoreType.DMA,
        ],
    )


def _neighbor_sum(atom_flat, adj_t, slots):
    return _neighbor_sum_kernel()(atom_flat, adj_t, slots)


TB = 1000
TR = 2000


def _p_body(bond_ref, sel_ref, p_ref):
    x = bond_ref[0]
    ssq = jnp.dot(x * x, sel_ref[...], preferred_element_type=jnp.float32)
    w = 1.0 / ssq
    w = w / jnp.maximum(jnp.sum(w, axis=1, keepdims=True), 1e-12)
    w8 = w[:, :8] * w[:, 8:]
    w4 = w8[:, :4] * w8[:, 4:]
    w2 = w4[:, :2] * w4[:, 2:]
    p_ref[...] = (w2[:, 0] * w2[:, 1]).reshape(1, 1, TB)


def _r_body(p_ref, r_ref):
    p = p_ref[...]
    q = p / jnp.maximum(jnp.sum(jnp.abs(p), axis=0, keepdims=True), 1e-12)
    s = 1.0 / q
    r_ref[...] = s / jnp.maximum(jnp.sum(jnp.abs(s), axis=1, keepdims=True), 1e-12)


def _out_body(a_ref, nb_ref, r_ref, w_ref, b_ref, o_ref):
    u = (a_ref[...] + nb_ref[...] * (1.0 / M)) * r_ref[...]
    y = jnp.dot(u, w_ref[...], preferred_element_type=jnp.float32) + b_ref[...]
    o_ref[...] = jnp.maximum(y, 0.0)


def kernel(atom, bond, adj_matrix, W1, W2, b1, b2):
    atom_flat = atom.reshape(B * N, F)

    adj = adj_matrix.astype(jnp.int32) + (jnp.arange(B, dtype=jnp.int32) * N)[:, None, None]
    adj_t = adj.reshape(B * N, M).T
    adj_t = jnp.pad(adj_t, ((0, 0), (0, NPAD - B * N)))
    slots = jnp.arange(NS * CHUNK, dtype=jnp.int32).reshape(NS, 1, CHUNK)

    neigh = _neighbor_sum(atom_flat, adj_t, slots)

    nblk = B * N // TB
    bond_r = bond.reshape(nblk, TB, M * FB)
    sel = jnp.kron(jnp.eye(M, dtype=jnp.float32),
                   jnp.ones((FB, 1), dtype=jnp.float32))
    p = pl.pallas_call(
        _p_body,
        out_shape=jax.ShapeDtypeStruct((nblk, 1, TB), jnp.float32),
        grid=(nblk,),
        in_specs=[
            pl.BlockSpec((1, TB, M * FB), lambda i: (i, 0, 0)),
            pl.BlockSpec((M * FB, M), lambda i: (0, 0)),
        ],
        out_specs=pl.BlockSpec((1, 1, TB), lambda i: (i, 0, 0)),
    )(bond_r, sel).reshape(B, N)

    r = pl.pallas_call(
        _r_body,
        out_shape=jax.ShapeDtypeStruct((B, N), jnp.float32),
    )(p)

    out = pl.pallas_call(
        _out_body,
        out_shape=jax.ShapeDtypeStruct((B * N, F), jnp.float32),
        grid=(B * N // TR,),
        in_specs=[
            pl.BlockSpec((TR, F), lambda i: (i, 0)),
            pl.BlockSpec((TR, F), lambda i: (i, 0)),
            pl.BlockSpec((TR, 1), lambda i: (i, 0)),
            pl.BlockSpec((F, F), lambda i: (0, 0)),
            pl.BlockSpec((1, F), lambda i: (0, 0)),
        ],
        out_specs=pl.BlockSpec((TR, F), lambda i: (i, 0)),
    )(atom_flat, neigh[: B * N], r.reshape(B * N, 1), W1, b1.reshape(1, F))

    return out.reshape(B, N, F)


# speedup vs baseline: 32.9934x; 2.9718x over previous
import functools

import jax
import jax.numpy as jnp
from jax import lax
from jax.experimental import pallas as pl
from jax.experimental.pallas import tpu as pltpu
from jax.experimental.pallas import tpu_sc as plsc

B, N, M = 2, 10000, 16
F = 128
FB = 16

NC, NS = 2, 16
NPAD = 20480
CHUNK = 128
W0_NODES = 640
W1_NODES = 640
NWAVE0 = W0_NODES // CHUNK
NWAVE1 = W1_NODES // CHUNK
CORE0_TOTAL = NS * W0_NODES

NSLOT = 3
WAVE = NS * CHUNK


def _neighbor_sum_body(atom_hbm, adj_hbm, slots_hbm, out_hbm,
                       idxd_v, slots_v, stage_v, acc,
                       gsem0, gsem1, gsem2, ssem0, ssem1, ssem2, isem):
    cid = lax.axis_index("c")
    sid = lax.axis_index("s")
    gsems = [gsem0, gsem1, gsem2]
    ssems = [ssem0, ssem1, ssem2]

    base_w = jnp.where(cid == 0, sid * W0_NODES, CORE0_TOTAL + sid * W1_NODES)
    nwave = jnp.where(cid == 0, NWAVE0, NWAVE1)
    my_slab = acc.at[pl.ds(sid * CHUNK, CHUNK)]

    pltpu.sync_copy(slots_hbm.at[sid], slots_v)
    pltpu.sync_copy(adj_hbm.at[:, pl.ds(base_w, CHUNK)], idxd_v.at[0])

    def wave(c, carry):
        ib = c % 2
        base = base_w + c * CHUNK

        @pl.when(c > 0)
        def _():
            pltpu.make_async_copy(adj_hbm.at[:, pl.ds(base_w, CHUNK)],
                                  idxd_v.at[ib], isem).wait()

        @pl.when(c + 1 < nwave)
        def _():
            pltpu.async_copy(adj_hbm.at[:, pl.ds(base + CHUNK, CHUNK)],
                             idxd_v.at[1 - ib], isem)

        def gather(m, slot, sem):
            return pltpu.async_copy(
                atom_hbm.at[idxd_v.at[ib, m]], stage_v.at[slot], sem)

        gather(0, 0, gsems[0]).wait()
        pltpu.sync_copy(stage_v.at[0], my_slab)

        for j in range(NSLOT):
            gather(j + 1, j, gsems[j])

        def step(i, carry2):
            for j in range(NSLOT):
                m = NSLOT * i + j + 1
                pltpu.make_async_copy(
                    atom_hbm.at[idxd_v.at[0, 0]],
                    stage_v.at[j], gsems[j]).wait()
                pltpu.async_copy(stage_v.at[j], acc.at[slots_v.at[0]],
                                 ssems[j], add=True).wait()
                gather(m + NSLOT, j, gsems[j])
            return carry2

        lax.fori_loop(0, (M - 4) // NSLOT, step, 0)

        for j in range(NSLOT):
            pltpu.make_async_copy(
                atom_hbm.at[idxd_v.at[0, 0]],
                stage_v.at[j], gsems[j]).wait()
            pltpu.async_copy(stage_v.at[j], acc.at[slots_v.at[0]],
                             ssems[j], add=True).wait()

        pltpu.sync_copy(my_slab, out_hbm.at[pl.ds(base, CHUNK)])
        return carry

    lax.fori_loop(0, nwave, wave, 0)


@functools.lru_cache(maxsize=None)
def _neighbor_sum_kernel():
    mesh = plsc.VectorSubcoreMesh(core_axis_name="c", subcore_axis_name="s")
    return pl.kernel(
        _neighbor_sum_body,
        out_type=jax.ShapeDtypeStruct((NPAD, F), jnp.float32),
        mesh=mesh,
        scratch_types=[
            pltpu.VMEM((2, M, CHUNK), jnp.int32),
            pltpu.VMEM((1, CHUNK), jnp.int32),
            pltpu.VMEM((NSLOT, CHUNK, F), jnp.float32),
            pltpu.VMEM_SHARED((WAVE, F), jnp.float32),
            pltpu.SemaphoreType.DMA,
            pltpu.SemaphoreType.DMA,
            pltpu.SemaphoreType.DMA,
            pltpu.SemaphoreType.DMA,
            pltpu.SemaphoreType.DMA,
            pltpu.SemaphoreType.DMA,
            pltpu.SemaphoreType.DMA,
        ],
    )


def _neighbor_sum(atom_flat, adj_t, slots):
    return _neighbor_sum_kernel()(atom_flat, adj_t, slots)


TB = 1000
TR = 2000


def _p_body(bond_ref, sel_ref, p_ref):
    x = bond_ref[0]
    ssq = jnp.dot(x * x, sel_ref[...], preferred_element_type=jnp.float32)
    w = 1.0 / ssq
    w = w / jnp.maximum(jnp.sum(w, axis=1, keepdims=True), 1e-12)
    w8 = w[:, :8] * w[:, 8:]
    w4 = w8[:, :4] * w8[:, 4:]
    w2 = w4[:, :2] * w4[:, 2:]
    p_ref[...] = (w2[:, 0] * w2[:, 1]).reshape(1, 1, TB)


def _r_body(p_ref, r_ref):
    p = p_ref[...]
    q = p / jnp.maximum(jnp.sum(jnp.abs(p), axis=0, keepdims=True), 1e-12)
    s = 1.0 / q
    r_ref[...] = s / jnp.maximum(jnp.sum(jnp.abs(s), axis=1, keepdims=True), 1e-12)


def _out_body(a_ref, nb_ref, r_ref, w_ref, b_ref, o_ref):
    u = (a_ref[...] + nb_ref[...] * (1.0 / M)) * r_ref[...]
    y = jnp.dot(u, w_ref[...], preferred_element_type=jnp.float32) + b_ref[...]
    o_ref[...] = jnp.maximum(y, 0.0)


def kernel(atom, bond, adj_matrix, W1, W2, b1, b2):
    atom_flat = atom.reshape(B * N, F)

    adj = adj_matrix.astype(jnp.int32) + (jnp.arange(B, dtype=jnp.int32) * N)[:, None, None]
    adj_t = adj.reshape(B * N, M).T
    npad = NPAD - B * N
    pad_idx = (jnp.arange(npad, dtype=jnp.int32)[None, :] * M
               + jnp.arange(M, dtype=jnp.int32)[:, None]) % (B * N)
    adj_t = jnp.concatenate([adj_t, pad_idx], axis=1)
    slots = jnp.arange(NS * CHUNK, dtype=jnp.int32).reshape(NS, 1, CHUNK)

    neigh = _neighbor_sum(atom_flat, adj_t, slots)

    nblk = B * N // TB
    bond_r = bond.reshape(nblk, TB, M * FB)
    sel = jnp.kron(jnp.eye(M, dtype=jnp.float32),
                   jnp.ones((FB, 1), dtype=jnp.float32))
    p = pl.pallas_call(
        _p_body,
        out_shape=jax.ShapeDtypeStruct((nblk, 1, TB), jnp.float32),
        grid=(nblk,),
        in_specs=[
            pl.BlockSpec((1, TB, M * FB), lambda i: (i, 0, 0)),
            pl.BlockSpec((M * FB, M), lambda i: (0, 0)),
        ],
        out_specs=pl.BlockSpec((1, 1, TB), lambda i: (i, 0, 0)),
    )(bond_r, sel).reshape(B, N)

    r = pl.pallas_call(
        _r_body,
        out_shape=jax.ShapeDtypeStruct((B, N), jnp.float32),
    )(p)

    out = pl.pallas_call(
        _out_body,
        out_shape=jax.ShapeDtypeStruct((B * N, F), jnp.float32),
        grid=(B * N // TR,),
        in_specs=[
            pl.BlockSpec((TR, F), lambda i: (i, 0)),
            pl.BlockSpec((TR, F), lambda i: (i, 0)),
            pl.BlockSpec((TR, 1), lambda i: (i, 0)),
            pl.BlockSpec((F, F), lambda i: (0, 0)),
            pl.BlockSpec((1, F), lambda i: (0, 0)),
        ],
        out_specs=pl.BlockSpec((TR, F), lambda i: (i, 0)),
    )(atom_flat, neigh, r.reshape(B * N, 1), W1, b1.reshape(1, F))

    return out.reshape(B, N, F)
